# baseline (speedup 1.0000x reference)
import functools

import jax
import jax.numpy as jnp
import numpy as np
from jax.experimental import pallas as pl

N = 10000
B = 8
NPG = N // B


def _mm_body(x_ref, w_ref, o_ref):
    o_ref[...] = jnp.dot(x_ref[...], w_ref[...],
                         preferred_element_type=jnp.float32)


def _mm(x, w):
    n, k = x.shape
    m = w.shape[1]
    bn = 512
    npad = ((n + bn - 1) // bn) * bn
    xp = jnp.pad(x, ((0, npad - n), (0, 0)))
    out = pl.pallas_call(
        _mm_body,
        grid=(npad // bn,),
        in_specs=[pl.BlockSpec((bn, k), lambda i: (i, 0)),
                  pl.BlockSpec((k, m), lambda i: (0, 0))],
        out_specs=pl.BlockSpec((bn, m), lambda i: (i, 0)),
        out_shape=jax.ShapeDtypeStruct((npad, m), jnp.float32),
    )(xp, w)
    return out[:n]


def _gcn(x, edge_index, W, b, num_nodes, edge_valid):
    src, dst = edge_index[0], edge_index[1]
    ew = jnp.ones((src.shape[0],), x.dtype) if edge_valid is None else edge_valid
    deg = jnp.zeros((num_nodes,), x.dtype).at[dst].add(ew)
    deg_safe = jnp.where(deg > 0, deg, 1.0)
    dinv = jnp.where(deg > 0, jax.lax.rsqrt(deg_safe), 0.0)
    norm = dinv[src] * ew * dinv[dst]
    h = _mm(x, W)
    out = jnp.zeros((num_nodes, W.shape[1]), x.dtype).at[dst].add(h[src] * norm[:, None])
    return out + b


def _bn(x, gamma, beta, eps=1e-5):
    mu = x.mean(axis=0)
    var = x.var(axis=0)
    return (x - mu) / jnp.sqrt(var + eps) * gamma + beta


def _score(x, edge_index, Wr, br, Wroot, edge_valid):
    src, dst = edge_index[0], edge_index[1]
    m = x[src]
    if edge_valid is not None:
        m = m * edge_valid[:, None]
    agg = jnp.zeros((x.shape[0], x.shape[1]), x.dtype).at[dst].add(m)
    return (agg @ Wr + br + x @ Wroot).reshape(-1)


def _pool(x, edge_index, edge_valid, n_per_graph, ratio, Wr, br, Wroot):
    num_nodes = x.shape[0]
    score = _score(x, edge_index, Wr, br, Wroot, edge_valid)
    k = int(np.ceil(ratio * n_per_graph))
    _, idx = jax.lax.top_k(score.reshape(B, n_per_graph), k)
    perm = (idx + jnp.arange(B, dtype=idx.dtype)[:, None] * n_per_graph).reshape(-1)
    x_new = x[perm] * jnp.tanh(score[perm])[:, None]
    node_map = jnp.full((num_nodes,), -1, dtype=jnp.int32).at[perm].set(
        jnp.arange(perm.shape[0], dtype=jnp.int32))
    src, dst = edge_index[0], edge_index[1]
    ns, nd = node_map[src], node_map[dst]
    valid = (ns >= 0) & (nd >= 0)
    if edge_valid is not None:
        valid = valid & (edge_valid > 0)
    ev = valid.astype(x.dtype)
    new_ei = jnp.stack([jnp.where(valid, ns, 0), jnp.where(valid, nd, 0)])
    return x_new, new_ei, ev, k


def kernel(x, edge_index, batch, y, W1, b1, W2, b2, W3, b3, W4, b4, g1, be1,
           g2, be2, g3, be3, p1Wr, p1br, p1Wroot, p2Wr, p2br, p2Wroot, p3Wr,
           p3br, p3Wroot, fcW, fcb):
    out = _gcn(x, edge_index, W1, b1, N, None)
    out = jax.nn.relu(_bn(out, g1, be1))
    out, ei, ev, n1 = _pool(out, edge_index, None, NPG, 0.6, p1Wr, p1br, p1Wroot)
    out = _gcn(out, ei, W2, b2, B * n1, ev)
    out = jax.nn.relu(_bn(out, g2, be2))
    out, ei, ev, n2 = _pool(out, ei, ev, n1, 0.6, p2Wr, p2br, p2Wroot)
    out = _gcn(out, ei, W3, b3, B * n2, ev)
    out = jax.nn.relu(_bn(out, g3, be3))
    out, ei, ev, n3 = _pool(out, ei, ev, n2, 0.5, p3Wr, p3br, p3Wroot)
    out = _gcn(out, ei, W4, b4, B * n3, ev)
    xr = out.reshape(B, n3, out.shape[1])
    feature = jnp.concatenate([xr.max(axis=1), xr.mean(axis=1)], axis=1)
    o = feature @ fcW + fcb
    return o, feature


# speedup vs baseline: 11.3433x; 11.3433x over previous
import functools

import jax
import jax.numpy as jnp
from jax import lax
from jax.experimental import pallas as pl
from jax.experimental.pallas import tpu as pltpu
from jax.experimental.pallas import tpu_sc as plsc

N = 10000
B = 8
NPG = N // B
E = 150000

NTILES = 32
EB = 128
NBATCH = 37
EPT = NBATCH * EB
E_PAD = NTILES * EPT

_MESH = plsc.VectorSubcoreMesh(core_axis_name="c", subcore_axis_name="s")

f32 = jnp.float32
i32 = jnp.int32


def _wid():
    return lax.axis_index("s") * 2 + lax.axis_index("c")


@functools.lru_cache(maxsize=None)
def _make_remap(n_prev, nperm, n_new, nacc):
    nmap = n_prev + 32
    trash_new = n_new

    def body(src_h, dst_h, perm_h, osn_h, odn_h, oso_h, odeg_h, ocnt_h,
             src_v, dst_v, perm_v, nmap_v, osn_v, odn_v, oso_v, deg_v, cnt_v):
        wid = _wid()
        e0 = pl.multiple_of(wid * EPT, 8)
        pltpu.sync_copy(src_h.at[pl.ds(e0, EPT)], src_v)
        pltpu.sync_copy(dst_h.at[pl.ds(e0, EPT)], dst_v)
        pltpu.sync_copy(perm_h, perm_v)

        def init_nmap(i, _):
            nmap_v[pl.ds(i * 16, 16)] = jnp.full((16,), -1, i32)
            return 0
        lax.fori_loop(0, nmap // 16, init_nmap, 0)

        def init_deg(i, _):
            deg_v[pl.ds(i * 16, 16)] = jnp.zeros((16,), f32)
            return 0
        lax.fori_loop(0, nacc // 16, init_deg, 0)

        def init_out(i, _):
            osn_v[pl.ds(i * 16, 16)] = jnp.zeros((16,), i32)
            odn_v[pl.ds(i * 16, 16)] = jnp.full((16,), trash_new, i32)
            oso_v[pl.ds(i * 16, 16)] = jnp.zeros((16,), i32)
            return 0
        lax.fori_loop(0, EPT // 16, init_out, 0)

        def scat_perm(i, _):
            pv = perm_v[pl.ds(i * 16, 16)]
            vals = jax.lax.iota(i32, 16) + i * 16
            plsc.store_scatter(nmap_v, [pv], vals)
            return 0
        lax.fori_loop(0, nperm // 16, scat_perm, 0)

        def edge_it(i, off):
            s = src_v[pl.ds(i * 16, 16)]
            d = dst_v[pl.ds(i * 16, 16)]
            ns = plsc.load_gather(nmap_v, [s])
            nd = plsc.load_gather(nmap_v, [d])
            valid = (ns >= 0) & (nd >= 0)
            dn = jnp.where(valid, nd, trash_new)
            sn = jnp.where(valid, ns, 0)
            so = jnp.where(valid, s, 0)
            plsc.addupdate_scatter(deg_v, [dn], jnp.ones((16,), f32))
            pos = plsc.cumsum(valid.astype(i32)) + off - 1
            plsc.store_scatter(osn_v, [pos], sn, mask=valid)
            plsc.store_scatter(odn_v, [pos], dn, mask=valid)
            plsc.store_scatter(oso_v, [pos], so, mask=valid)
            return off + plsc.all_reduce_population_count(valid)
        off = lax.fori_loop(0, EPT // 16, edge_it, jnp.zeros((16,), i32))

        cnt_v[...] = off
        pltpu.sync_copy(osn_v, osn_h.at[pl.ds(e0, EPT)])
        pltpu.sync_copy(odn_v, odn_h.at[pl.ds(e0, EPT)])
        pltpu.sync_copy(oso_v, oso_h.at[pl.ds(e0, EPT)])
        pltpu.sync_copy(deg_v, odeg_h.at[wid])
        pltpu.sync_copy(cnt_v, ocnt_h.at[wid])

    return pl.kernel(
        body,
        out_type=(
            jax.ShapeDtypeStruct((E_PAD,), i32),
            jax.ShapeDtypeStruct((E_PAD,), i32),
            jax.ShapeDtypeStruct((E_PAD,), i32),
            jax.ShapeDtypeStruct((NTILES, nacc), f32),
            jax.ShapeDtypeStruct((NTILES, 16), i32),
        ),
        mesh=_MESH,
        compiler_params=pltpu.CompilerParams(needs_layout_passes=False),
        scratch_types=[
            pltpu.VMEM((EPT,), i32),
            pltpu.VMEM((EPT,), i32),
            pltpu.VMEM((nperm,), i32),
            pltpu.VMEM((nmap,), i32),
            pltpu.VMEM((EPT,), i32),
            pltpu.VMEM((EPT,), i32),
            pltpu.VMEM((EPT,), i32),
            pltpu.VMEM((nacc,), f32),
            pltpu.VMEM((16,), i32),
        ],
    )


@functools.lru_cache(maxsize=None)
def _make_agg(nrows, nacc, nc):
    rpt = nacc // 16

    def body(*refs):
        tables = refs[:nc]
        src_h, dst_h, cnt_h = refs[nc:nc + 3]
        outs = refs[nc + 3:nc + 3 + 2 * nc]
        src_v, dst_v, cnt_v, rows_v, zbuf, acc, sem = refs[nc + 3 + 2 * nc:]
        cid = lax.axis_index("c")
        sid = lax.axis_index("s")
        wid = sid * 2 + cid
        pltpu.sync_copy(src_h.at[wid], src_v)
        pltpu.sync_copy(dst_h.at[wid], dst_v)
        pltpu.sync_copy(cnt_h.at[wid], cnt_v)
        cnt = jnp.max(cnt_v[...])
        nb = (cnt + (EB - 1)) // EB

        for zr in range(64):
            for zc in range(8):
                zbuf[zr, pl.ds(zc * 16, 16)] = jnp.zeros((16,), f32)

        r0 = sid * rpt
        for c in range(nc):
            for kk in range(rpt // 64):
                pltpu.sync_copy(zbuf, acc.at[pl.ds(r0 + kk * 64, 64)])
            plsc.subcore_barrier()

            def batch_it(b, _):
                pltpu.async_copy(tables[c].at[src_v.at[b]], rows_v, sem).wait()
                pltpu.sync_copy(rows_v, acc.at[dst_v.at[b]], add=True)
                return 0
            lax.fori_loop(0, nb, batch_it, 0, unroll=False)
            plsc.subcore_barrier()

            for cc in range(2):
                @pl.when(cid == cc)
                def _():
                    for kk in range(rpt // 64):
                        pltpu.sync_copy(
                            acc.at[pl.ds(r0 + kk * 64, 64)],
                            outs[cc * nc + c].at[pl.ds(r0 + kk * 64, 64)])
            plsc.subcore_barrier()

    return pl.kernel(
        body,
        out_type=tuple(jax.ShapeDtypeStruct((nacc, 128), f32)
                       for _ in range(2 * nc)),
        mesh=_MESH,
        compiler_params=pltpu.CompilerParams(needs_layout_passes=False),
        scratch_types=[
            pltpu.VMEM((NBATCH, EB), i32),
            pltpu.VMEM((NBATCH, EB), i32),
            pltpu.VMEM((16,), i32),
            pltpu.VMEM((EB, 128), f32),
            pltpu.VMEM((64, 128), f32),
            pltpu.VMEM_SHARED((nacc, 128), f32),
            pltpu.SemaphoreType.DMA,
        ],
    )


@functools.lru_cache(maxsize=None)
def _make_topk(npg, npg_pad, k, kpad):
    nv = npg_pad // 16

    def body(score_h, perm_h, ssel_h, sc_v, key_v, perm_v, ssel_v):
        wid = _wid()

        @pl.when(wid < B)
        def _():
            g = wid
            pltpu.sync_copy(score_h.at[g], sc_v)

            def mkkey(i, _):
                x = sc_v[pl.ds(i * 16, 16)]
                xi = plsc.bitcast(x, i32)
                m = jax.lax.shift_right_arithmetic(xi, 31) | jnp.int32(-2147483648)
                key_v[pl.ds(i * 16, 16)] = plsc.bitcast(xi ^ m, jnp.uint32)
                return 0
            lax.fori_loop(0, nv, mkkey, 0)

            thr = jnp.zeros((16,), jnp.uint32)
            for bit in range(31, -1, -1):
                cand = thr | jnp.uint32(1 << bit)

                def cnt_it(i, acc):
                    kv = key_v[pl.ds(i * 16, 16)]
                    return acc + jnp.where(kv >= cand, 1, 0).astype(i32)
                cvec = lax.fori_loop(0, nv, cnt_it, jnp.zeros((16,), i32))
                total = jnp.sum(cvec)
                thr = jnp.where(total >= k, cand, thr)

            def init_sel(i, _):
                perm_v[pl.ds(i * 16, 16)] = jnp.zeros((16,), i32)
                ssel_v[pl.ds(i * 16, 16)] = jnp.zeros((16,), f32)
                return 0
            lax.fori_loop(0, kpad // 16, init_sel, 0)

            def cp_it(i, off):
                kv = key_v[pl.ds(i * 16, 16)]
                xv = sc_v[pl.ds(i * 16, 16)]
                m = kv >= thr
                pos = plsc.cumsum(m.astype(i32)) + off
                wm = m & (pos <= k)
                gidx = jax.lax.iota(i32, 16) + (i * 16 + g * npg)
                plsc.store_scatter(perm_v, [pos - 1], gidx, mask=wm)
                plsc.store_scatter(ssel_v, [pos - 1], xv, mask=wm)
                return off + plsc.all_reduce_population_count(m)
            lax.fori_loop(0, nv, cp_it, jnp.zeros((16,), i32))

            pltpu.sync_copy(perm_v, perm_h.at[g])
            pltpu.sync_copy(ssel_v, ssel_h.at[g])

    return pl.kernel(
        body,
        out_type=(
            jax.ShapeDtypeStruct((B, kpad), i32),
            jax.ShapeDtypeStruct((B, kpad), f32),
        ),
        mesh=_MESH,
        compiler_params=pltpu.CompilerParams(needs_layout_passes=False),
        scratch_types=[
            pltpu.VMEM((npg_pad,), f32),
            pltpu.VMEM((npg_pad,), jnp.uint32),
            pltpu.VMEM((kpad,), i32),
            pltpu.VMEM((kpad,), f32),
        ],
    )


_BN = 512


@functools.lru_cache(maxsize=None)
def _make_mm(nrows, kdim, dout):
    nc = dout // 128

    def body(x_ref, w_ref, pre_ref, post_ref, *o_refs):
        x = x_ref[...] * pre_ref[...]
        h = jnp.dot(x, w_ref[...], preferred_element_type=f32)
        h = h * post_ref[...]
        for c in range(nc):
            o_refs[c][...] = h[:, c * 128:(c + 1) * 128]

    grid = (nrows // _BN,)
    return pl.pallas_call(
        body,
        grid=grid,
        in_specs=[
            pl.BlockSpec((_BN, kdim), lambda i: (i, 0)),
            pl.BlockSpec((kdim, dout), lambda i: (0, 0)),
            pl.BlockSpec((_BN, 1), lambda i: (i, 0)),
            pl.BlockSpec((_BN, 1), lambda i: (i, 0)),
        ],
        out_specs=[pl.BlockSpec((_BN, 128), lambda i: (i, 0))
                   for _ in range(nc)],
        out_shape=[jax.ShapeDtypeStruct((nrows, 128), f32)
                   for _ in range(nc)],
    )


@functools.lru_cache(maxsize=None)
def _make_bn_a(nacc, d, n):
    nc = d // 128

    def body(*refs):
        parts = refs[:2 * nc]
        deg_ref, b_ref = refs[2 * nc:2 * nc + 2]
        y_ref, st_ref = refs[2 * nc + 2:]
        i = pl.program_id(0)
        deg = jnp.sum(deg_ref[...], axis=0)
        deg_safe = jnp.where(deg > 0, deg, 1.0)
        dinv = jnp.where(deg > 0, lax.rsqrt(deg_safe), 0.0)
        agg = jnp.concatenate(
            [parts[c][...] + parts[nc + c][...] for c in range(nc)], axis=1)
        y = agg * dinv[:, None] + b_ref[...]
        y_ref[...] = y
        rowid = i * _BN + lax.broadcasted_iota(i32, (_BN, 1), 0)
        mask = rowid < n
        ym = jnp.where(mask, y, 0.0)
        s1 = jnp.sum(ym, axis=0, keepdims=True)
        s2 = jnp.sum(ym * ym, axis=0, keepdims=True)
        upd = jnp.concatenate([s1, s2, jnp.zeros((6, d), f32)], axis=0)

        @pl.when(i == 0)
        def _():
            st_ref[...] = jnp.zeros((8, d), f32)
        st_ref[...] += upd

    grid = (nacc // _BN,)
    in_specs = ([pl.BlockSpec((_BN, 128), lambda i: (i, 0))] * (2 * nc)
                + [pl.BlockSpec((NTILES, _BN), lambda i: (0, i)),
                   pl.BlockSpec((1, d), lambda i: (0, 0))])
    return pl.pallas_call(
        body,
        grid=grid,
        in_specs=in_specs,
        out_specs=[pl.BlockSpec((_BN, d), lambda i: (i, 0)),
                   pl.BlockSpec((8, d), lambda i: (0, 0))],
        out_shape=[jax.ShapeDtypeStruct((nacc, d), f32),
                   jax.ShapeDtypeStruct((8, d), f32)],
    )


@functools.lru_cache(maxsize=None)
def _make_bn_b(nacc, d, n):
    nc = d // 128

    def body(y_ref, st_ref, g_ref, be_ref, flat_ref, *c_refs):
        mu = st_ref[0:1, :] / n
        var = st_ref[1:2, :] / n - mu * mu
        out = (y_ref[...] - mu) / jnp.sqrt(var + 1e-5) * g_ref[...] + be_ref[...]
        out = jnp.maximum(out, 0.0)
        flat_ref[...] = out
        for c in range(nc):
            c_refs[c][...] = out[:, c * 128:(c + 1) * 128]

    grid = (nacc // _BN,)
    return pl.pallas_call(
        body,
        grid=grid,
        in_specs=[
            pl.BlockSpec((_BN, d), lambda i: (i, 0)),
            pl.BlockSpec((8, d), lambda i: (0, 0)),
            pl.BlockSpec((1, d), lambda i: (0, 0)),
            pl.BlockSpec((1, d), lambda i: (0, 0)),
        ],
        out_specs=([pl.BlockSpec((_BN, d), lambda i: (i, 0))]
                   + [pl.BlockSpec((_BN, 128), lambda i: (i, 0))] * nc),
        out_shape=([jax.ShapeDtypeStruct((nacc, d), f32)]
                   + [jax.ShapeDtypeStruct((nacc, 128), f32)] * nc),
    )


@functools.lru_cache(maxsize=None)
def _make_asm(nacc, d):
    nc = d // 128

    def body(*refs):
        parts = refs[:2 * nc]
        o_ref = refs[2 * nc]
        o_ref[...] = jnp.concatenate(
            [parts[c][...] + parts[nc + c][...] for c in range(nc)], axis=1)

    grid = (nacc // _BN,)
    return pl.pallas_call(
        body,
        grid=grid,
        in_specs=[pl.BlockSpec((_BN, 128), lambda i: (i, 0))] * (2 * nc),
        out_specs=pl.BlockSpec((_BN, d), lambda i: (i, 0)),
        out_shape=jax.ShapeDtypeStruct((nacc, d), f32),
    )


@functools.lru_cache(maxsize=None)
def _make_final(nacc, d, npg_f, n):
    nc = d // 128

    def body(*refs):
        parts = refs[:2 * nc]
        deg_ref, b_ref = refs[2 * nc:2 * nc + 2]
        f_ref = refs[2 * nc + 2]
        deg = jnp.sum(deg_ref[...], axis=0)
        deg_safe = jnp.where(deg > 0, deg, 1.0)
        dinv = jnp.where(deg > 0, lax.rsqrt(deg_safe), 0.0)
        agg = jnp.concatenate(
            [parts[c][...] + parts[nc + c][...] for c in range(nc)], axis=1)
        y = agg * dinv[:, None] + b_ref[...]
        rowid = lax.broadcasted_iota(i32, (nacc, 1), 0)
        gid = rowid // npg_f
        maxs = []
        means = []
        for g in range(B):
            m = (gid == g) & (rowid < n)
            mx = jnp.max(jnp.where(m, y, -jnp.inf), axis=0, keepdims=True)
            mn = jnp.sum(jnp.where(m, y, 0.0), axis=0, keepdims=True) / npg_f
            maxs.append(mx)
            means.append(mn)
        f_ref[...] = jnp.concatenate(
            [jnp.concatenate(maxs, axis=0), jnp.concatenate(means, axis=0)],
            axis=1)

    return pl.pallas_call(
        body,
        grid=(1,),
        in_specs=([pl.BlockSpec((nacc, 128), lambda i: (0, 0))] * (2 * nc)
                  + [pl.BlockSpec((NTILES, nacc), lambda i: (0, 0)),
                     pl.BlockSpec((1, d), lambda i: (0, 0))]),
        out_specs=pl.BlockSpec((B, 2 * d), lambda i: (0, 0)),
        out_shape=jax.ShapeDtypeStruct((B, 2 * d), f32),
    )


def _dinv_np(deg_parts, n):
    deg = jnp.sum(deg_parts, axis=0)[:n]
    return jnp.where(deg > 0, lax.rsqrt(jnp.where(deg > 0, deg, 1.0)), 0.0)


@jax.jit
def _forward_impl(x, edge_index, W1, b1, W2, b2, W3, b3, W4, b4, g1, be1, g2,
                  be2, g3, be3, p1Wr, p1br, p1Wroot, p2Wr, p2br, p2Wroot,
                  p3Wr, p3br, p3Wroot, fcW, fcb):
    NACC1, NACC2, NACC3, NACC4 = 10240, 6144, 4096, 2048

    src = edge_index[0]
    dst = edge_index[1]
    pad = E_PAD - E
    src_p = jnp.concatenate([src, jnp.zeros((pad,), i32)])
    dst_p = jnp.concatenate([dst, jnp.full((pad,), N, i32)])

    def e3d(a):
        return a.reshape(NTILES, NBATCH, EB)

    remap1 = _make_remap(N, N, N, NACC1)
    sn1, dn1, so1, degp1, cnt1 = remap1(
        src_p, dst_p, jnp.arange(N, dtype=i32))
    dinv1 = _dinv_np(degp1, N)
    post1 = jnp.pad(dinv1, (0, NACC1 - N)).reshape(NACC1, 1)
    ones1 = jnp.ones((NACC1, 1), f32)
    xp = jnp.pad(x, ((0, NACC1 - N), (0, 0)))
    h1 = _make_mm(NACC1, 512, 512)(xp, W1, ones1, post1)
    agg1 = _make_agg(NACC1, NACC1, 4)(*h1, e3d(so1), e3d(dn1), cnt1)
    y1, st1 = _make_bn_a(NACC1, 512, N)(*agg1, degp1, b1.reshape(1, 512))
    bnb1 = _make_bn_b(NACC1, 512, N)(y1, st1, g1.reshape(1, 512),
                                     be1.reshape(1, 512))
    out1_flat, out1c = bnb1[0], bnb1[1:]

    sagg1 = _make_agg(NACC1, NACC1, 4)(*out1c, e3d(sn1), e3d(dn1), cnt1)
    sagg1f = _make_asm(NACC1, 512)(*sagg1)
    score1 = (sagg1f[:N] @ p1Wr + p1br + out1_flat[:N] @ p1Wroot).reshape(-1)
    sc2d = jnp.pad(score1.reshape(B, NPG), ((0, 0), (0, 1264 - NPG)),
                   constant_values=-jnp.inf)
    perm2d, ssel2d = _make_topk(NPG, 1264, 750, 752)(sc2d)
    perm1 = perm2d[:, :750].reshape(-1)
    t1 = jnp.tanh(ssel2d[:, :750].reshape(-1))
    n1 = B * 750

    remap2 = _make_remap(N, n1, n1, NACC2)
    sn2, dn2, so2, degp2, cnt2 = remap2(sn1, dn1, perm1)
    dinv2 = _dinv_np(degp2, n1)
    pre2 = jnp.zeros((NACC1,), f32).at[perm1].set(t1).reshape(NACC1, 1)
    post2 = jnp.zeros((NACC1,), f32).at[perm1].set(dinv2).reshape(NACC1, 1)
    h2 = _make_mm(NACC1, 512, 512)(out1_flat, W2, pre2, post2)
    agg2 = _make_agg(NACC1, NACC2, 4)(*h2, e3d(so2), e3d(dn2), cnt2)
    y2, st2 = _make_bn_a(NACC2, 512, n1)(*agg2, degp2, b2.reshape(1, 512))
    bnb2 = _make_bn_b(NACC2, 512, n1)(y2, st2, g2.reshape(1, 512),
                                      be2.reshape(1, 512))
    out2_flat, out2c = bnb2[0], bnb2[1:]

    sagg2 = _make_agg(NACC2, NACC2, 4)(*out2c, e3d(sn2), e3d(dn2), cnt2)
    sagg2f = _make_asm(NACC2, 512)(*sagg2)
    score2 = (sagg2f[:n1] @ p2Wr + p2br + out2_flat[:n1] @ p2Wroot).reshape(-1)
    sc2d2 = jnp.pad(score2.reshape(B, 750), ((0, 0), (0, 752 - 750)),
                    constant_values=-jnp.inf)
    perm2d2, ssel2d2 = _make_topk(750, 752, 450, 456)(sc2d2)
    perm2 = perm2d2[:, :450].reshape(-1)
    t2 = jnp.tanh(ssel2d2[:, :450].reshape(-1))
    n2 = B * 450

    remap3 = _make_remap(n1, n2, n2, NACC3)
    sn3, dn3, so3, degp3, cnt3 = remap3(sn2, dn2, perm2)
    dinv3 = _dinv_np(degp3, n2)
    pre3 = jnp.zeros((NACC2,), f32).at[perm2].set(t2).reshape(NACC2, 1)
    post3 = jnp.zeros((NACC2,), f32).at[perm2].set(dinv3).reshape(NACC2, 1)
    h3 = _make_mm(NACC2, 512, 256)(out2_flat, W3, pre3, post3)
    agg3 = _make_agg(NACC2, NACC3, 2)(*h3, e3d(so3), e3d(dn3), cnt3)
    y3, st3 = _make_bn_a(NACC3, 256, n2)(*agg3, degp3, b3.reshape(1, 256))
    bnb3 = _make_bn_b(NACC3, 256, n2)(y3, st3, g3.reshape(1, 256),
                                      be3.reshape(1, 256))
    out3_flat, out3c = bnb3[0], bnb3[1:]

    sagg3 = _make_agg(NACC3, NACC3, 2)(*out3c, e3d(sn3), e3d(dn3), cnt3)
    sagg3f = _make_asm(NACC3, 256)(*sagg3)
    score3 = (sagg3f[:n2] @ p3Wr + p3br + out3_flat[:n2] @ p3Wroot).reshape(-1)
    sc2d3 = jnp.pad(score3.reshape(B, 450), ((0, 0), (0, 464 - 450)),
                    constant_values=-jnp.inf)
    perm2d3, ssel2d3 = _make_topk(450, 464, 225, 232)(sc2d3)
    perm3 = perm2d3[:, :225].reshape(-1)
    t3 = jnp.tanh(ssel2d3[:, :225].reshape(-1))
    n3 = B * 225

    permp = jnp.concatenate([perm3, jnp.full((8,), n2 + 16, i32)])
    remap4 = _make_remap(n2, n3 + 8, n3, NACC4)
    sn4, dn4, so4, degp4, cnt4 = remap4(sn3, dn3, permp)
    dinv4 = _dinv_np(degp4, n3)
    pre4 = jnp.zeros((NACC3,), f32).at[perm3].set(t3).reshape(NACC3, 1)
    post4 = jnp.zeros((NACC3,), f32).at[perm3].set(dinv4).reshape(NACC3, 1)
    h4 = _make_mm(NACC3, 256, 256)(out3_flat, W4, pre4, post4)
    agg4 = _make_agg(NACC3, NACC4, 2)(*h4, e3d(so4), e3d(dn4), cnt4)
    feature = _make_final(NACC4, 256, 225, n3)(*agg4, degp4,
                                               b4.reshape(1, 256))
    o = feature @ fcW + fcb
    return o, feature


def kernel(x, edge_index, batch, y, W1, b1, W2, b2, W3, b3, W4, b4, g1, be1,
           g2, be2, g3, be3, p1Wr, p1br, p1Wroot, p2Wr, p2br, p2Wroot, p3Wr,
           p3br, p3Wroot, fcW, fcb):
    return _forward_impl(x, edge_index, W1, b1, W2, b2, W3, b3, W4, b4, g1,
                         be1, g2, be2, g3, be3, p1Wr, p1br, p1Wroot, p2Wr,
                         p2br, p2Wroot, p3Wr, p3br, p3Wroot, fcW, fcb)
